# Initial kernel scaffold; baseline (speedup 1.0000x reference)
import jax
import jax.numpy as jnp
from jax.experimental import pallas as pl


def kernel(latent, cond, emb, ce_W1, ce_b1, ce_W2, ce_b2, lin_W, lin_b, n0_w, n0_b, c0_W1, c0_b1, c0_W2, c0_b2, u0_W1, u0_b1, u0_W2, u0_b2, n1_w, n1_b, c1_W1, c1_b1, c1_W2, c1_b2):
    raise NotImplementedError("write your pallas kernel here")


# speedup vs baseline: 10.3845x; 10.3845x over previous
import functools

import jax
import jax.numpy as jnp
from jax import lax
from jax.experimental import pallas as pl
from jax.experimental.pallas import tpu as pltpu
from jax.experimental.pallas import tpu_sc as plsc

B = 8
LC = 128
H = 64
CD = 128
UP0 = 256
UP1 = 4
K = 16
NCLS = 55
P0 = UP0
P1 = UP0 * UP1

_NW = 32


def _elu(x):
    return jnp.where(x > 0, x, jnp.exp(x) - 1.0)


def _bdot(a, b):
    return jnp.dot(a.astype(jnp.bfloat16), b.astype(jnp.bfloat16),
                   preferred_element_type=jnp.float32)


def _front_body(latent_ref, cond_ref, emb_ref, w1_ref, b1_ref, w2_ref, b2_ref,
                linw_ref, linb_ref, n0w_ref, n0b_ref, c_out_ref, h_out_ref):
    cond = cond_ref[...]
    oh = (cond == lax.broadcasted_iota(jnp.int32, (B, NCLS), 1)).astype(jnp.float32)
    c = jnp.dot(oh, emb_ref[...], preferred_element_type=jnp.float32,
                precision=jax.lax.Precision.HIGHEST)
    c = _elu(c)
    c = _elu(_bdot(c, w1_ref[...]) + b1_ref[...])
    c = _bdot(c, w2_ref[...]) + b2_ref[...]
    c_out_ref[...] = c
    z = jnp.concatenate([latent_ref[...], c], axis=1)
    h = _bdot(z, linw_ref[...]) + linb_ref[...]
    m = jnp.mean(h, axis=1, keepdims=True)
    d0 = h - m
    v = jnp.mean(d0 * d0, axis=1, keepdims=True)
    h_out_ref[...] = d0 / jnp.sqrt(v + 1e-5) * n0w_ref[...] + n0b_ref[...]


def _front(latent, cond2, emb, ce_W1, ce_b1, ce_W2, ce_b2, lin_W, lin_b,
           n0w_t, n0b_t):
    return pl.pallas_call(
        _front_body,
        out_shape=(
            jax.ShapeDtypeStruct((B, CD), jnp.float32),
            jax.ShapeDtypeStruct((B, P0 * H), jnp.float32),
        ),
    )(latent, cond2, emb, ce_W1, ce_b1.reshape(1, -1), ce_W2,
      ce_b2.reshape(1, -1), lin_W, lin_b.reshape(1, -1), n0w_t, n0b_t)


def _gln1_body(h_ref, w_ref, b_ref, out_ref):
    h = h_ref[...]
    m = jnp.mean(h, axis=1, keepdims=True)
    d0 = h - m
    v = jnp.mean(d0 * d0, axis=1, keepdims=True)
    out_ref[...] = _elu(d0 / jnp.sqrt(v + 1e-5) * w_ref[...] + b_ref[...])


def _gln1(h_flat, w_t, b_t):
    return pl.pallas_call(
        _gln1_body,
        out_shape=jax.ShapeDtypeStruct((B, P1 * H), jnp.float32),
    )(h_flat, w_t, b_t)


def _knn_body(p, h_ref, idx_ref):
    b = pl.program_id(0)
    x = h_ref[0]
    sq = jnp.sum(x * x, axis=1)
    xb = x.astype(jnp.bfloat16)
    d = (sq[:, None] + sq[None, :]
         - 2.0 * lax.dot_general(xb, xb, (((1,), (1,)), ((), ())),
                                 preferred_element_type=jnp.float32))
    rows = lax.broadcasted_iota(jnp.int32, (p, p), 0)
    cols = lax.broadcasted_iota(jnp.int32, (p, p), 1)
    d = jnp.where(rows == cols, d + 1e9, d)
    sel_rows = []
    for _ in range(K):
        m = jnp.min(d, axis=1, keepdims=True)
        sel = jnp.min(jnp.where(d <= m, cols, p), axis=1)
        sel_rows.append(sel)
        d = jnp.where(cols == sel[:, None], jnp.float32(jnp.inf), d)
    idx_ref[0] = jnp.stack(sel_rows, axis=0) + b * p


def _knn(h3, p):
    return pl.pallas_call(
        functools.partial(_knn_body, p),
        grid=(B,),
        in_specs=[pl.BlockSpec((1, p, H), lambda b: (b, 0, 0))],
        out_specs=pl.BlockSpec((1, K, p), lambda b: (b, 0, 0)),
        out_shape=jax.ShapeDtypeStruct((B, K, p), jnp.int32),
    )(h3)


def _make_sc_gather(e_rows, n_rows):
    nc = 2
    rpw = e_rows // _NW
    ch = min(1024, rpw)
    n_chunks = rpw // ch
    n_sub = ch // 128
    mesh = plsc.VectorSubcoreMesh(core_axis_name="c", subcore_axis_name="s",
                                  num_cores=nc, num_subcores=_NW // nc)

    @functools.partial(
        pl.kernel,
        mesh=mesh,
        compiler_params=pltpu.CompilerParams(use_tc_tiling_on_sc=False),
        out_type=jax.ShapeDtypeStruct((e_rows, H), jnp.float32),
        scratch_types=[
            pltpu.VMEM((n_sub, 128), jnp.int32),
            pltpu.VMEM((ch, H), jnp.float32),
            pltpu.SemaphoreType.DMA,
        ],
    )
    def gather(h_hbm, idx_hbm, out_hbm, idx_v, rows_v, sem):
        wid = lax.axis_index("s") * nc + lax.axis_index("c")
        for cidx in range(n_chunks):
            base = pl.multiple_of(wid * rpw + cidx * ch, ch)
            pltpu.sync_copy(
                idx_hbm.at[pl.ds(pl.multiple_of(base // 128, n_sub), n_sub)],
                idx_v)
            handles = [
                pltpu.async_copy(h_hbm.at[idx_v.at[j]],
                                 rows_v.at[pl.ds(j * 128, 128)], sem)
                for j in range(n_sub)
            ]
            for hd in handles:
                hd.wait()
            pltpu.sync_copy(rows_v, out_hbm.at[pl.ds(base, ch)])

    del n_rows
    return gather


def _conv_body(apply_tanh, h_ref, hj_ref, c_ref, w1_ref, b1_ref, w2_ref,
               b2_ref, out_ref):
    x = h_ref[0]
    w1h = w1_ref[0:H, :]
    w1c = w1_ref[H:H + CD, :]
    w1d = w1_ref[H + CD:2 * H + CD, :].astype(jnp.bfloat16)
    w2 = w2_ref[...].astype(jnp.bfloat16)
    pre = (_bdot(x, w1h)
           + _bdot(c_ref[0], w1c)
           + b1_ref[...])
    acc = None
    for k in range(K):
        dj = (hj_ref[0, k] - x).astype(jnp.bfloat16)
        e = _elu(pre + jnp.dot(dj, w1d, preferred_element_type=jnp.float32))
        o = (jnp.dot(e.astype(jnp.bfloat16), w2,
                     preferred_element_type=jnp.float32) + b2_ref[...])
        acc = o if acc is None else jnp.maximum(acc, o)
    out_ref[0] = jnp.tanh(acc) if apply_tanh else acc


def _conv(h3, hj, c, W1, b1, W2, b2, p, apply_tanh=False):
    dout = W2.shape[1]
    wspec = lambda shape: pl.BlockSpec(shape, lambda b: tuple(0 for _ in shape))
    return pl.pallas_call(
        functools.partial(_conv_body, apply_tanh),
        grid=(B,),
        in_specs=[
            pl.BlockSpec((1, p, H), lambda b: (b, 0, 0)),
            pl.BlockSpec((1, K, p, H), lambda b: (b, 0, 0, 0)),
            pl.BlockSpec((1, 1, CD), lambda b: (b, 0, 0)),
            wspec(W1.shape), wspec((1, 2 * H)), wspec(W2.shape),
            wspec((1, dout)),
        ],
        out_specs=pl.BlockSpec((1, p, dout), lambda b: (b, 0, 0)),
        out_shape=jax.ShapeDtypeStruct((B, p, dout), jnp.float32),
    )(h3, hj, c.reshape(B, 1, CD), W1, b1.reshape(1, -1), W2,
      b2.reshape(1, -1))


def kernel(latent, cond, emb, ce_W1, ce_b1, ce_W2, ce_b2, lin_W, lin_b, n0_w,
           n0_b, c0_W1, c0_b1, c0_W2, c0_b2, u0_W1, u0_b1, u0_W2, u0_b2, n1_w,
           n1_b, c1_W1, c1_b1, c1_W2, c1_b2):
    cond2 = cond.astype(jnp.int32).reshape(B, 1)
    n0w_t = jnp.tile(n0_w, P0).reshape(1, -1)
    n0b_t = jnp.tile(n0_b, P0).reshape(1, -1)
    n1w_t = jnp.tile(n1_w, P1).reshape(1, -1)
    n1b_t = jnp.tile(n1_b, P1).reshape(1, -1)

    c, h0f = _front(latent, cond2, emb, ce_W1, ce_b1, ce_W2, ce_b2,
                    lin_W, lin_b, n0w_t, n0b_t)
    h0 = h0f.reshape(B, P0, H)

    gather_p0 = _make_sc_gather(B * K * P0, B * P0)
    idx0 = _knn(h0, P0)
    hj0 = gather_p0(h0f.reshape(B * P0, H),
                    idx0.reshape(-1, 128)).reshape(B, K, P0, H)
    hc = _conv(h0, hj0, c, c0_W1, c0_b1, c0_W2, c0_b2, P0)

    idx0b = _knn(hc, P0)
    hjb = gather_p0(hc.reshape(B * P0, H),
                    idx0b.reshape(-1, 128)).reshape(B, K, P0, H)
    hu = _conv(hc, hjb, c, u0_W1, u0_b1, u0_W2, u0_b2, P0)

    h1f = _gln1(jnp.concatenate([hc, hu], axis=-1).reshape(B, P1 * H),
                n1w_t, n1b_t)
    h1 = h1f.reshape(B, P1, H)

    idx1 = _knn(h1, P1)
    hj1 = _make_sc_gather(B * K * P1, B * P1)(
        h1f.reshape(B * P1, H), idx1.reshape(-1, 128)).reshape(B, K, P1, H)

    out = _conv(h1, hj1, c, c1_W1, c1_b1, c1_W2, c1_b2, P1,
                apply_tanh=True)
    return out.reshape(B * P1, 3)
